# baseline (speedup 1.0000x reference)
import functools
import jax
import jax.numpy as jnp
from jax import lax
from jax.experimental import pallas as pl
from jax.experimental.pallas import tpu as pltpu
from jax.experimental.pallas import tpu_sc as plsc

_N = 8192
_NUMEL = _N * _N
_NW = 32
_E = _NUMEL // _NW
_CHUNK = 16384
_NCHUNKS = _E // _CHUNK
_VECS = _CHUNK // 16
_UNROLL = 8

_mesh = plsc.VectorSubcoreMesh(core_axis_name="c", subcore_axis_name="s")


@functools.partial(
    pl.kernel,
    mesh=_mesh,
    out_type=jax.ShapeDtypeStruct((_NW, 16), jnp.float32),
    scratch_types=[
        pltpu.VMEM((_CHUNK,), jnp.float32),
        pltpu.VMEM((_CHUNK,), jnp.float32),
        pltpu.VMEM((_CHUNK,), jnp.float32),
        pltpu.VMEM((_CHUNK,), jnp.float32),
        pltpu.VMEM((16,), jnp.float32),
        pltpu.SemaphoreType.DMA,
        pltpu.SemaphoreType.DMA,
    ],
)
def _sc_body(pred, target, out, pb0, tb0, pb1, tb1, accbuf, sem0, sem1):
    c = lax.axis_index("c")
    s = lax.axis_index("s")
    wid = s * 2 + c
    base = wid * _E

    def start(buf_p, buf_t, sem, off):
        pltpu.async_copy(pred.at[pl.ds(off, _CHUNK)], buf_p, sem)
        pltpu.async_copy(target.at[pl.ds(off, _CHUNK)], buf_t, sem)

    def wait(buf_p, buf_t, sem):
        pltpu.make_async_copy(pred.at[pl.ds(0, _CHUNK)], buf_p, sem).wait()
        pltpu.make_async_copy(target.at[pl.ds(0, _CHUNK)], buf_t, sem).wait()

    def consume(buf_p, buf_t, vacc):
        def inner(i, acc):
            for u in range(_UNROLL):
                idx = i * (16 * _UNROLL) + u * 16
                p = buf_p[pl.ds(idx, 16)]
                t = buf_t[pl.ds(idx, 16)]
                d = jnp.minimum(jnp.maximum(p, 0.0), 1.0) - t
                acc = acc + d * d
            return acc

        return lax.fori_loop(0, _VECS // _UNROLL, inner, vacc)

    start(pb0, tb0, sem0, base)

    def body(g, vacc):
        start(pb1, tb1, sem1, base + (2 * g + 1) * _CHUNK)
        wait(pb0, tb0, sem0)
        vacc = consume(pb0, tb0, vacc)

        @pl.when(g < _NCHUNKS // 2 - 1)
        def _():
            start(pb0, tb0, sem0, base + (2 * g + 2) * _CHUNK)

        wait(pb1, tb1, sem1)
        vacc = consume(pb1, tb1, vacc)
        return vacc

    vacc = lax.fori_loop(0, _NCHUNKS // 2, body, jnp.zeros((16,), jnp.float32))
    accbuf[...] = vacc
    pltpu.sync_copy(accbuf, out.at[wid])


def kernel(pred, target):
    out = _sc_body(pred.reshape(-1), target.reshape(-1))
    return jnp.sum(out) * (1.0 / float(_NUMEL))


# speedup vs baseline: 1.0008x; 1.0008x over previous
import functools
import jax
import jax.numpy as jnp
from jax import lax
from jax.experimental import pallas as pl
from jax.experimental.pallas import tpu as pltpu
from jax.experimental.pallas import tpu_sc as plsc

_N = 8192
_NUMEL = _N * _N
_NW = 32
_E = _NUMEL // _NW
_CHUNK = 16384
_NCHUNKS = _E // _CHUNK
_VECS = _CHUNK // 16
_UNROLL = 16

_mesh = plsc.VectorSubcoreMesh(core_axis_name="c", subcore_axis_name="s")


@functools.partial(
    pl.kernel,
    mesh=_mesh,
    out_type=jax.ShapeDtypeStruct((_NW, 16), jnp.float32),
    scratch_types=[
        pltpu.VMEM((_CHUNK,), jnp.float32),
        pltpu.VMEM((_CHUNK,), jnp.float32),
        pltpu.VMEM((_CHUNK,), jnp.float32),
        pltpu.VMEM((_CHUNK,), jnp.float32),
        pltpu.VMEM((16,), jnp.float32),
        pltpu.SemaphoreType.DMA,
        pltpu.SemaphoreType.DMA,
    ],
)
def _sc_body(pred, target, out, pb0, tb0, pb1, tb1, accbuf, sem0, sem1):
    c = lax.axis_index("c")
    s = lax.axis_index("s")
    wid = s * 2 + c
    base = wid * _E

    def start(buf_p, buf_t, sem, off):
        pltpu.async_copy(pred.at[pl.ds(off, _CHUNK)], buf_p, sem)
        pltpu.async_copy(target.at[pl.ds(off, _CHUNK)], buf_t, sem)

    def wait(buf_p, buf_t, sem):
        pltpu.make_async_copy(pred.at[pl.ds(0, _CHUNK)], buf_p, sem).wait()
        pltpu.make_async_copy(target.at[pl.ds(0, _CHUNK)], buf_t, sem).wait()

    def consume(buf_p, buf_t, accs):
        @plsc.parallel_loop(0, _VECS, step=_UNROLL, unroll=2, carry=accs)
        def inner(i, acc_t):
            a0, a1, a2, a3 = acc_t
            acc4 = [a0, a1, a2, a3]
            for u in range(_UNROLL):
                idx = (i + u) * 16
                p = buf_p[pl.ds(idx, 16)]
                t = buf_t[pl.ds(idx, 16)]
                d = jnp.minimum(jnp.maximum(p, 0.0), 1.0) - t
                acc4[u % 4] = acc4[u % 4] + d * d
            return tuple(acc4)

        return inner

    start(pb0, tb0, sem0, base)

    zero = jnp.zeros((16,), jnp.float32)

    def body(g, accs):
        start(pb1, tb1, sem1, base + (2 * g + 1) * _CHUNK)
        wait(pb0, tb0, sem0)
        accs = consume(pb0, tb0, accs)

        @pl.when(g < _NCHUNKS // 2 - 1)
        def _():
            start(pb0, tb0, sem0, base + (2 * g + 2) * _CHUNK)

        wait(pb1, tb1, sem1)
        accs = consume(pb1, tb1, accs)
        return accs

    accs = lax.fori_loop(0, _NCHUNKS // 2, body, (zero, zero, zero, zero))
    accbuf[...] = (accs[0] + accs[1]) + (accs[2] + accs[3])
    pltpu.sync_copy(accbuf, out.at[wid])


def kernel(pred, target):
    out = _sc_body(pred.reshape(-1), target.reshape(-1))
    return jnp.sum(out) * (1.0 / float(_NUMEL))
